# baseline (speedup 1.0000x reference)
import functools

import jax
import jax.numpy as jnp
from jax import lax
from jax.experimental import pallas as pl
from jax.experimental.pallas import tpu as pltpu
from jax.experimental.pallas import tpu_sc as plsc

F = 26
V = 100000
D = 32
B = 4096
EPS = 1e-5

NC = 2
NS = 16
NW = NC * NS
RW = (B * F) // NW
CH = 128
NCH = RW // CH
NB = RW // 16


def _rsqrt(x):
    i = plsc.bitcast(x, jnp.int32)
    i = jnp.int32(0x5F3759DF) - (i >> 1)
    y = plsc.bitcast(i, jnp.float32)
    for _ in range(3):
        y = y * (1.5 - 0.5 * x * y * y)
    return y


@functools.partial(
    pl.kernel,
    out_type=jax.ShapeDtypeStruct((B * F, D), jnp.float32),
    mesh=plsc.VectorSubcoreMesh(
        core_axis_name="c", subcore_axis_name="s", num_cores=NC, num_subcores=NS
    ),
    compiler_params=pltpu.CompilerParams(
        use_tc_tiling_on_sc=False, needs_layout_passes=False
    ),
    scratch_types=[
        pltpu.VMEM((RW,), jnp.int32),
        pltpu.VMEM((RW, D), jnp.float32),
        pltpu.VMEM((D,), jnp.float32),
        pltpu.VMEM((D,), jnp.float32),
        pltpu.SemaphoreType.DMA,
    ],
)
def _sc_embed_ln(x_hbm, tab_hbm, gamma_hbm, beta_hbm, out_hbm,
                 idx_v, rows_v, g_v, b_v, sem):
    wid = lax.axis_index("s") * NC + lax.axis_index("c")
    base = wid * RW

    pltpu.sync_copy(gamma_hbm, g_v)
    pltpu.sync_copy(beta_hbm, b_v)

    pltpu.sync_copy(x_hbm.at[pl.ds(base, RW)], idx_v)

    iota = lax.iota(jnp.int32, 16)

    def fire(j, carry):
        for kk in range(CH // 16):
            s = pl.ds(j * CH + kk * 16, 16)
            pos = iota + (j * CH + kk * 16) + base
            idx_v[s] = idx_v[s] + (pos % F) * V
        pltpu.async_copy(
            tab_hbm.at[idx_v.at[pl.ds(j * CH, CH)]],
            rows_v.at[pl.ds(j * CH, CH)],
            sem,
        )
        return carry

    lax.fori_loop(0, NCH, fire, 0)

    def drain(j, carry):
        pltpu.make_async_copy(
            tab_hbm.at[idx_v.at[pl.ds(j * CH, CH)]],
            rows_v.at[pl.ds(j * CH, CH)],
            sem,
        ).wait()
        return carry

    lax.fori_loop(0, NCH, drain, 0)

    def ln_block(bi, carry):
        ids = iota + bi * 16
        vs = []
        acc = jnp.zeros((16,), jnp.float32)
        acc2 = jnp.zeros((16,), jnp.float32)
        for d in range(D):
            col = jnp.full((16,), d, jnp.int32)
            v = plsc.load_gather(rows_v, [ids, col])
            vs.append(v)
            acc = acc + v
            acc2 = acc2 + v * v
        mean = acc * (1.0 / D)
        var = acc2 * (1.0 / D) - mean * mean
        r = _rsqrt(var + EPS)
        for d in range(D):
            col = jnp.full((16,), d, jnp.int32)
            g = plsc.load_gather(g_v, [col])
            b = plsc.load_gather(b_v, [col])
            o = (vs[d] - mean) * r * g + b
            plsc.store_scatter(rows_v, [ids, col], o)
        return carry

    lax.fori_loop(0, NB, ln_block, 0)

    pltpu.sync_copy(rows_v, out_hbm.at[pl.ds(base, RW)])


def kernel(x, tables, gamma, beta):
    x_flat = x.reshape(-1).astype(jnp.int32)
    tab = tables.reshape(F * V, D)
    out = _sc_embed_ln(x_flat, tab,
                       gamma.astype(jnp.float32), beta.astype(jnp.float32))
    return out.reshape(B, F, D)
